# baseline (speedup 1.0000x reference)
import functools

import jax
import jax.numpy as jnp
from jax import lax
from jax.experimental import pallas as pl
from jax.experimental.pallas import tpu as pltpu
from jax.experimental.pallas import tpu_sc as plsc

VOCAB_SIZE = 1000000
EMBEDDING_DIM = 32
BATCH = 16384
SEQ_LEN = 50

_NC = 2
_NS = 16
_NW = _NC * _NS

_BPT = BATCH // _NW
_CH = 128
_CPS = _BPT // _CH
_STEPS = SEQ_LEN * _CPS
_NBUF = 4
_NT = 2
_LANES = 16
_VPC = _CH // _LANES


def _body(table_hbm, idxT_hbm, out_hbm, idx_v, rows_v, tbuf, gsems, osems):
    wid = lax.axis_index("s") * _NC + lax.axis_index("c")
    b0 = wid * _BPT

    def split(step):
        return lax.div(step, _CPS), lax.rem(step, _CPS)

    def gather(slot, step):
        s, ch = split(step)
        return pltpu.make_async_copy(
            table_hbm.at[idx_v.at[s, pl.ds(ch * _CH, _CH)]],
            rows_v.at[slot], gsems.at[slot])

    def outcopy(ts, step):
        s, ch = split(step)
        return pltpu.make_async_copy(
            tbuf.at[ts],
            out_hbm.at[s, :, pl.ds(b0 + ch * _CH, _CH)],
            osems.at[ts])

    bvecs = [v * _LANES + lax.iota(jnp.int32, _LANES) for v in range(_VPC)]

    def transpose(slot, ts, step):
        rows = rows_v.at[slot]
        s, ch = split(step)
        zv = [idx_v[s, pl.ds(ch * _CH + v * _LANES, _LANES)] & 0
              for v in range(_VPC)]
        for c in range(EMBEDDING_DIM):
            for v in range(_VPC):
                vals = plsc.load_gather(rows, [bvecs[v], zv[v] + c])
                tbuf[ts, c, pl.ds(v * _LANES, _LANES)] = vals

    pltpu.sync_copy(idxT_hbm.at[:, pl.ds(b0, _BPT)], idx_v)

    for step in range(_NBUF):
        gather(step, step).start()

    def group(g, carry):
        for k in range(_NBUF):
            step = g * _NBUF + k
            ts = k % _NT
            gather(k, step).wait()

            @pl.when(step >= _NT)
            def _():
                outcopy(ts, step - _NT).wait()

            transpose(k, ts, step)
            outcopy(ts, step).start()

            @pl.when(step + _NBUF < _STEPS)
            def _():
                gather(k, step + _NBUF).start()

        return carry

    lax.fori_loop(0, _STEPS // _NBUF, group, 0)

    for step in (_STEPS - 2, _STEPS - 1):
        outcopy(step % _NT, step).wait()


@jax.jit
def _embedding_gather(idxT, table):
    mesh = plsc.VectorSubcoreMesh(core_axis_name="c", subcore_axis_name="s")
    run = pl.kernel(
        _body,
        out_type=jax.ShapeDtypeStruct((SEQ_LEN, EMBEDDING_DIM, BATCH),
                                      jnp.float32),
        mesh=mesh,
        scratch_types=[
            pltpu.VMEM((SEQ_LEN, _BPT), jnp.int32),
            pltpu.VMEM((_NBUF, _CH, EMBEDDING_DIM), jnp.float32),
            pltpu.VMEM((_NT, EMBEDDING_DIM, _CH), jnp.float32),
            pltpu.SemaphoreType.DMA((_NBUF,)),
            pltpu.SemaphoreType.DMA((_NT,)),
        ],
        compiler_params=pltpu.CompilerParams(use_tc_tiling_on_sc=False,
                                             needs_layout_passes=False),
    )
    return run(table, idxT)


def kernel(inputs, shared_weights):
    idxT = inputs.astype(jnp.int32).T
    out3 = _embedding_gather(idxT, shared_weights)
    return out3.transpose(2, 0, 1)


# speedup vs baseline: 1.1323x; 1.1323x over previous
import functools

import jax
import jax.numpy as jnp
from jax import lax
from jax.experimental import pallas as pl
from jax.experimental.pallas import tpu as pltpu
from jax.experimental.pallas import tpu_sc as plsc

VOCAB_SIZE = 1000000
EMBEDDING_DIM = 32
BATCH = 16384
SEQ_LEN = 50

_NC = 2
_NS = 16
_NW = _NC * _NS

_BPT = BATCH // _NW
_CH = 128
_CPS = _BPT // _CH
_STEPS = SEQ_LEN * _CPS
_NBUF = 4
_NT = 2
_LANES = 16
_VPC = _CH // _LANES


def _body(table_hbm, idxT_hbm, out_hbm, idx_v, rows_v, tbuf, gsems, osems):
    wid = lax.axis_index("s") * _NC + lax.axis_index("c")
    b0 = wid * _BPT

    def split(step):
        return lax.div(step, _CPS), lax.rem(step, _CPS)

    def gather(slot, step):
        s, ch = split(step)
        return pltpu.make_async_copy(
            table_hbm.at[idx_v.at[s, pl.ds(ch * _CH, _CH)]],
            rows_v.at[slot], gsems.at[slot])

    def outcopy(ts, step):
        s, ch = split(step)
        return pltpu.make_async_copy(
            tbuf.at[ts],
            out_hbm.at[s, :, pl.ds(b0 + ch * _CH, _CH)],
            osems.at[ts])

    bvecs = [v * _LANES + lax.iota(jnp.int32, _LANES) for v in range(_VPC)]

    def transpose(slot, ts, step):
        rows = rows_v.at[slot]
        s, ch = split(step)
        zv = [idx_v[s, pl.ds(ch * _CH + v * _LANES, _LANES)] & 0
              for v in range(_VPC)]
        for c in range(EMBEDDING_DIM):
            vals = [plsc.load_gather(rows, [bvecs[v], zv[v] + c])
                    for v in range(_VPC)]
            for v in range(_VPC):
                tbuf[ts, c, pl.ds(v * _LANES, _LANES)] = vals[v]

    pltpu.sync_copy(idxT_hbm.at[:, pl.ds(b0, _BPT)], idx_v)

    for step in range(_NBUF):
        gather(step, step).start()

    def group(g, carry):
        for k in range(_NBUF):
            step = g * _NBUF + k
            ts = k % _NT
            gather(k, step).wait()

            @pl.when(step >= _NT)
            def _():
                outcopy(ts, step - _NT).wait()

            transpose(k, ts, step)
            outcopy(ts, step).start()

            @pl.when(step + _NBUF < _STEPS)
            def _():
                gather(k, step + _NBUF).start()

        return carry

    lax.fori_loop(0, _STEPS // _NBUF, group, 0)

    for step in (_STEPS - 2, _STEPS - 1):
        outcopy(step % _NT, step).wait()


@jax.jit
def _embedding_gather(idxT, table):
    mesh = plsc.VectorSubcoreMesh(core_axis_name="c", subcore_axis_name="s")
    run = pl.kernel(
        _body,
        out_type=jax.ShapeDtypeStruct((SEQ_LEN, EMBEDDING_DIM, BATCH),
                                      jnp.float32),
        mesh=mesh,
        scratch_types=[
            pltpu.VMEM((SEQ_LEN, _BPT), jnp.int32),
            pltpu.VMEM((_NBUF, _CH, EMBEDDING_DIM), jnp.float32),
            pltpu.VMEM((_NT, EMBEDDING_DIM, _CH), jnp.float32),
            pltpu.SemaphoreType.DMA((_NBUF,)),
            pltpu.SemaphoreType.DMA((_NT,)),
        ],
        compiler_params=pltpu.CompilerParams(use_tc_tiling_on_sc=False,
                                             needs_layout_passes=False),
    )
    return run(table, idxT)


def kernel(inputs, shared_weights):
    idxT = inputs.astype(jnp.int32).T
    out3 = _embedding_gather(idxT, shared_weights)
    return out3.transpose(2, 0, 1)


# speedup vs baseline: 1.3384x; 1.1820x over previous
import functools

import jax
import jax.numpy as jnp
from jax import lax
from jax.experimental import pallas as pl
from jax.experimental.pallas import tpu as pltpu
from jax.experimental.pallas import tpu_sc as plsc

VOCAB_SIZE = 1000000
EMBEDDING_DIM = 32
BATCH = 16384
SEQ_LEN = 50

_NC = 2
_NS = 16
_NW = _NC * _NS

_BPT = BATCH // _NW
_NBUF = 4
_NT = 2
_LANES = 16


def _body(table_hbm, idxT_hbm, out_hbm, idx_v, rows_v, gsems, osems):
    wid = lax.axis_index("s") * _NC + lax.axis_index("c")
    b0 = wid * _BPT

    def gather(slot, s):
        return pltpu.make_async_copy(table_hbm.at[idx_v.at[s]],
                                     rows_v.at[slot], gsems.at[slot])

    def outcopy(slot, s):
        return pltpu.make_async_copy(rows_v.at[slot],
                                     out_hbm.at[s, pl.ds(b0, _BPT), :],
                                     osems.at[slot])

    pltpu.sync_copy(idxT_hbm.at[:, pl.ds(b0, _BPT)], idx_v)

    for s in range(_NBUF):
        gather(s, s).start()

    def group(g, carry):
        for k in range(_NBUF):
            s = g * _NBUF + k

            @pl.when(s < SEQ_LEN)
            def _():
                gather(k, s).wait()
                outcopy(k, s).start()

                @pl.when(s + _NBUF < SEQ_LEN)
                def _():
                    outcopy(k, s).wait()
                    gather(k, s + _NBUF).start()

        return carry

    lax.fori_loop(0, (SEQ_LEN + _NBUF - 1) // _NBUF, group, 0)

    for k in range(_NBUF):
        outcopy(k, SEQ_LEN - _NBUF + k).wait()


@jax.jit
def _embedding_gather(idxT, table):
    mesh = plsc.VectorSubcoreMesh(core_axis_name="c", subcore_axis_name="s")
    run = pl.kernel(
        _body,
        out_type=jax.ShapeDtypeStruct((SEQ_LEN, BATCH, EMBEDDING_DIM),
                                      jnp.float32),
        mesh=mesh,
        scratch_types=[
            pltpu.VMEM((SEQ_LEN, _BPT), jnp.int32),
            pltpu.VMEM((_NBUF, _BPT, EMBEDDING_DIM), jnp.float32),
            pltpu.SemaphoreType.DMA((_NBUF,)),
            pltpu.SemaphoreType.DMA((_NBUF,)),
        ],
        compiler_params=pltpu.CompilerParams(use_tc_tiling_on_sc=False,
                                             needs_layout_passes=False),
    )
    return run(table, idxT)


def kernel(inputs, shared_weights):
    idxT = inputs.astype(jnp.int32).T
    out3 = _embedding_gather(idxT, shared_weights)
    return out3.transpose(1, 0, 2)
